# Initial kernel scaffold; baseline (speedup 1.0000x reference)
import jax
import jax.numpy as jnp
from jax.experimental import pallas as pl


def kernel(node_attr, edge_index, edge_attr, eb_W0, eb_b0, eb_W1, eb_b1, eb_W2, eb_b2, eb_W3, eb_b3, nb_W0, nb_b0, nb_W1, nb_b1, nb_W2, nb_b2, nb_W3, nb_b3):
    raise NotImplementedError("write your pallas kernel here")


# speedup vs baseline: 3.1467x; 3.1467x over previous
import functools

import jax
import jax.numpy as jnp
from jax import lax
from jax.experimental import pallas as pl
from jax.experimental.pallas import tpu as pltpu
from jax.experimental.pallas import tpu_sc as plsc

NW = 32


def _sc_gather(node_attr, edge_index):
    n, h = node_attr.shape
    e = edge_index.shape[1]
    c = 256
    k = c // 128
    n_chunks = e // c
    iters = (n_chunks + NW - 1) // NW

    mesh = plsc.VectorSubcoreMesh(core_axis_name="c", subcore_axis_name="s")

    @functools.partial(
        pl.kernel,
        out_type=(
            jax.ShapeDtypeStruct((e, h), jnp.float32),
            jax.ShapeDtypeStruct((e, h), jnp.float32),
        ),
        mesh=mesh,
        scratch_types=[
            pltpu.VMEM((k, 128), jnp.int32),
            pltpu.VMEM((k, 128), jnp.int32),
            pltpu.VMEM((c, h), jnp.float32),
            pltpu.VMEM((c, h), jnp.float32),
            pltpu.SemaphoreType.DMA,
        ],
    )
    def gather_kernel(node_hbm, ei_hbm, sg_hbm, rg_hbm, sidx, ridx, srows, rrows, sem):
        wid = lax.axis_index("s") * 2 + lax.axis_index("c")

        @pl.loop(0, iters)
        def _(it):
            chunk = it * NW + wid

            @pl.when(chunk < n_chunks)
            def _():
                base = chunk * c
                for j in range(k):
                    pltpu.sync_copy(ei_hbm.at[0, pl.ds(base + j * 128, 128)], sidx.at[j])
                    pltpu.sync_copy(ei_hbm.at[1, pl.ds(base + j * 128, 128)], ridx.at[j])
                copies = []
                for j in range(k):
                    copies.append(
                        pltpu.async_copy(node_hbm.at[sidx.at[j]], srows.at[pl.ds(j * 128, 128), :], sem)
                    )
                    copies.append(
                        pltpu.async_copy(node_hbm.at[ridx.at[j]], rrows.at[pl.ds(j * 128, 128), :], sem)
                    )
                for cp in copies:
                    cp.wait()
                pltpu.sync_copy(srows, sg_hbm.at[pl.ds(base, c), :])
                pltpu.sync_copy(rrows, rg_hbm.at[pl.ds(base, c), :])

    return gather_kernel(node_attr, edge_index)


def _sc_scatter(edge_msg, edge_index, zeros_nh):
    e, h = edge_msg.shape
    n_pad = zeros_nh.shape[0]
    c = 256
    k = c // 128
    n_chunks = e // c
    iters = (n_chunks + NW - 1) // NW
    rows_per_tile = n_pad // 16

    mesh = plsc.VectorSubcoreMesh(core_axis_name="c", subcore_axis_name="s")

    @functools.partial(
        pl.kernel,
        out_type=jax.ShapeDtypeStruct((2, n_pad, h), jnp.float32),
        mesh=mesh,
        scratch_types=[
            pltpu.VMEM((k, 128), jnp.int32),
            pltpu.VMEM((c, h), jnp.float32),
            pltpu.VMEM_SHARED((n_pad, h), jnp.float32),
        ],
    )
    def scatter_kernel(msg_hbm, ei_hbm, z_hbm, out_hbm, ridx, rows, agg):
        cid = lax.axis_index("c")
        sid = lax.axis_index("s")
        wid = sid * 2 + cid
        pltpu.sync_copy(
            z_hbm.at[pl.ds(sid * rows_per_tile, rows_per_tile), :],
            agg.at[pl.ds(sid * rows_per_tile, rows_per_tile), :],
        )
        plsc.subcore_barrier()

        @pl.loop(0, iters)
        def _(it):
            chunk = it * NW + wid

            @pl.when(chunk < n_chunks)
            def _():
                base = chunk * c
                pltpu.sync_copy(msg_hbm.at[pl.ds(base, c), :], rows)
                for j in range(k):
                    pltpu.sync_copy(ei_hbm.at[1, pl.ds(base + j * 128, 128)], ridx.at[j])
                for j in range(k):
                    pltpu.sync_copy(rows.at[pl.ds(j * 128, 128), :], agg.at[ridx.at[j]], add=True)

        plsc.subcore_barrier()
        pltpu.sync_copy(
            agg.at[pl.ds(sid * rows_per_tile, rows_per_tile), :],
            out_hbm.at[cid, pl.ds(sid * rows_per_tile, rows_per_tile), :],
        )

    return scatter_kernel(edge_msg, edge_index, zeros_nh)


def _tc_edge_mlp(sg, rg, ea, w0, b0, w1, b1, w2, b2, w3, b3):
    e, h = ea.shape
    t = 1280
    grid = (e // t,)

    def body(sg_ref, rg_ref, ea_ref, w0_ref, b0_ref, w1_ref, b1_ref, w2_ref,
             b2_ref, w3_ref, b3_ref, en_ref, eo_ref):
        ea_v = ea_ref[...]
        acc = (
            jnp.dot(sg_ref[...], w0_ref[0:h, :], preferred_element_type=jnp.float32)
            + jnp.dot(rg_ref[...], w0_ref[h:2 * h, :], preferred_element_type=jnp.float32)
            + jnp.dot(ea_v, w0_ref[2 * h:3 * h, :], preferred_element_type=jnp.float32)
            + b0_ref[...]
        )
        acc = jnp.maximum(acc, 0.0)
        acc = jnp.maximum(jnp.dot(acc, w1_ref[...], preferred_element_type=jnp.float32) + b1_ref[...], 0.0)
        acc = jnp.maximum(jnp.dot(acc, w2_ref[...], preferred_element_type=jnp.float32) + b2_ref[...], 0.0)
        en = jnp.dot(acc, w3_ref[...], preferred_element_type=jnp.float32) + b3_ref[...]
        en_ref[...] = en
        eo_ref[...] = ea_v + en

    full = lambda shape: pl.BlockSpec(shape, lambda i: (0,) * len(shape))
    tile = pl.BlockSpec((t, h), lambda i: (i, 0))
    return pl.pallas_call(
        body,
        grid=grid,
        in_specs=[
            tile, tile, tile,
            full((3 * h, h)), full((1, h)),
            full((h, h)), full((1, h)),
            full((h, h)), full((1, h)),
            full((h, h)), full((1, h)),
        ],
        out_specs=[tile, tile],
        out_shape=[
            jax.ShapeDtypeStruct((e, h), jnp.float32),
            jax.ShapeDtypeStruct((e, h), jnp.float32),
        ],
    )(sg, rg, ea, w0, b0.reshape(1, h), w1, b1.reshape(1, h),
      w2, b2.reshape(1, h), w3, b3.reshape(1, h))


def _tc_node_mlp(x, p0, p1, w0, b0, w1, b1, w2, b2, w3, b3):
    n, h = x.shape
    t = 1000
    grid = (n // t,)

    def body(x_ref, p0_ref, p1_ref, w0_ref, b0_ref, w1_ref, b1_ref, w2_ref,
             b2_ref, w3_ref, b3_ref, xo_ref):
        x_v = x_ref[...]
        agg = p0_ref[...] + p1_ref[...]
        acc = (
            jnp.dot(x_v, w0_ref[0:h, :], preferred_element_type=jnp.float32)
            + jnp.dot(agg, w0_ref[h:2 * h, :], preferred_element_type=jnp.float32)
            + b0_ref[...]
        )
        acc = jnp.maximum(acc, 0.0)
        acc = jnp.maximum(jnp.dot(acc, w1_ref[...], preferred_element_type=jnp.float32) + b1_ref[...], 0.0)
        acc = jnp.maximum(jnp.dot(acc, w2_ref[...], preferred_element_type=jnp.float32) + b2_ref[...], 0.0)
        xo_ref[...] = x_v + jnp.dot(acc, w3_ref[...], preferred_element_type=jnp.float32) + b3_ref[...]

    full = lambda shape: pl.BlockSpec(shape, lambda i: (0,) * len(shape))
    tile = pl.BlockSpec((t, h), lambda i: (i, 0))
    return pl.pallas_call(
        body,
        grid=grid,
        in_specs=[
            tile, tile, tile,
            full((2 * h, h)), full((1, h)),
            full((h, h)), full((1, h)),
            full((h, h)), full((1, h)),
            full((h, h)), full((1, h)),
        ],
        out_specs=tile,
        out_shape=jax.ShapeDtypeStruct((n, h), jnp.float32),
    )(x, p0, p1, w0, b0.reshape(1, h), w1, b1.reshape(1, h),
      w2, b2.reshape(1, h), w3, b3.reshape(1, h))


def kernel(node_attr, edge_index, edge_attr,
           eb_W0, eb_b0, eb_W1, eb_b1, eb_W2, eb_b2, eb_W3, eb_b3,
           nb_W0, nb_b0, nb_W1, nb_b1, nb_W2, nb_b2, nb_W3, nb_b3):
    n, h = node_attr.shape
    sg, rg = _sc_gather(node_attr, edge_index)
    en, edge_out = _tc_edge_mlp(sg, rg, edge_attr,
                                eb_W0, eb_b0, eb_W1, eb_b1, eb_W2, eb_b2, eb_W3, eb_b3)
    n_pad = ((n + 127) // 128) * 128
    zeros = jnp.zeros((n_pad, h), jnp.float32)
    partials = _sc_scatter(en, edge_index, zeros)
    x_out = _tc_node_mlp(node_attr, partials[0, :n], partials[1, :n],
                         nb_W0, nb_b0, nb_W1, nb_b1, nb_W2, nb_b2, nb_W3, nb_b3)
    return (x_out, edge_out)
